# baseline (speedup 1.0000x reference)
import dataclasses

import jax
import jax.numpy as jnp
from jax import lax
from jax.experimental import pallas as pl
from jax.experimental import pallas as pl_sc
from jax.experimental.pallas import tpu as pltpu
from jax.experimental.pallas import tpu_sc as plsc

_N = 10000
_E = 320000
_D = 128
_BE = 1280

_NW = 32
_PER_W = _E // _NW
_CH = 80
_NCH = _PER_W // _CH


_GW = 64


def _sc_gather(ah, bh, src, dst):
    mesh = plsc.VectorSubcoreMesh(core_axis_name="c", subcore_axis_name="s")
    out = jax.ShapeDtypeStruct((_E, _D), jnp.float32)

    src2 = src.reshape(_E // _GW, 1, _GW)
    dst2 = dst.reshape(_E // _GW, 1, _GW)

    @pl.kernel(
        mesh=mesh,
        out_type=[out, out],
        scratch_types=[
            pltpu.VMEM_SHARED((_N, _D), jnp.float32),
            pltpu.VMEM((_GW, _D), jnp.float32),
        ],
    )
    def k(ah_hbm, bh_hbm, src_hbm, dst_hbm, ga_hbm, gsum_hbm, bh_sh, tmp):
        sid = lax.axis_index("s")

        rows = 624
        pltpu.sync_copy(bh_hbm.at[pl.ds(sid * rows, rows)],
                        bh_sh.at[pl.ds(sid * rows, rows)])

        @pl.when(sid == 0)
        def _():
            pltpu.sync_copy(bh_hbm.at[pl.ds(16 * rows, _N - 16 * rows)],
                            bh_sh.at[pl.ds(16 * rows, _N - 16 * rows)])

        plsc.subcore_barrier()

        def body(src_v, dst_v, ga_v, gsum_v):
            pltpu.sync_copy(ah_hbm.at[src_v.at[0, 0]], ga_v)
            pltpu.sync_copy(bh_sh.at[src_v.at[0, 0]], gsum_v)
            pltpu.sync_copy(bh_sh.at[dst_v.at[0, 0]], tmp)

            @pl.loop(0, _GW)
            def _(rr):
                for c in range(_D // 16):
                    sl = pl.ds(c * 16, 16)
                    gsum_v[rr, sl] = gsum_v[rr, sl] + tmp[rr, sl]

        pltpu.emit_pipeline(
            body,
            grid=(_E // _GW,),
            in_specs=[
                pl.BlockSpec((1, 1, _GW), lambda i: (i, 0, 0)),
                pl.BlockSpec((1, 1, _GW), lambda i: (i, 0, 0)),
            ],
            out_specs=[
                pl.BlockSpec((_GW, _D), lambda i: (i, 0)),
                pl.BlockSpec((_GW, _D), lambda i: (i, 0)),
            ],
            core_axis_name=("c", "s"),
            dimension_semantics=(pltpu.PARALLEL,),
        )(src_hbm, dst_hbm, ga_hbm, gsum_hbm)

    return k(ah, bh, src2, dst2)


_CHE = 640
_EH = _E // 2
_NCHE = _EH // _CHE
_RPW = 8


def _sc_segmax(msgt, dst):
    mesh = plsc.VectorSubcoreMesh(core_axis_name="c", subcore_axis_name="s")

    cp = pltpu.CompilerParams()
    if "needs_layout_passes" in pltpu.CompilerParams.__dataclass_fields__:
        cp = dataclasses.replace(cp, needs_layout_passes=False)

    @pl.kernel(
        mesh=mesh,
        compiler_params=cp,
        out_type=jax.ShapeDtypeStruct((2 * _D * _N,), jnp.float32),
        scratch_types=[
            pltpu.VMEM((_RPW * _N,), jnp.float32),
            pltpu.VMEM((_N,), jnp.int32),
            pltpu.VMEM((_CHE,), jnp.int32),
            pltpu.VMEM((_CHE,), jnp.int32),
            pltpu.VMEM((_RPW, _CHE), jnp.float32),
            pltpu.VMEM((_RPW, _CHE), jnp.float32),
            pltpu.SemaphoreType.DMA,
            pltpu.SemaphoreType.DMA,
        ],
    )
    def k(msgt_hbm, dst_hbm, ct_hbm, acc, tmpn, dv0, dv1, mv0, mv1,
          sem0, sem1):
        w = lax.axis_index("s") * 2 + lax.axis_index("c")
        half = w % 2
        g = w // 2
        r0 = g * _RPW
        e0 = half * _EH

        neg_inf = jnp.full((16,), -jnp.inf, dtype=jnp.float32)
        io = lax.iota(jnp.int32, 16)
        io16 = io + 16

        @pl.loop(0, _RPW * _N, step=16)
        def _(cc):
            acc[pl.ds(cc, 16)] = neg_inf

        def start(ci, dv, mv, sem):
            off = e0 + ci * _CHE
            pltpu.async_copy(dst_hbm.at[pl.ds(off, _CHE)], dv, sem)
            pltpu.async_copy(
                msgt_hbm.at[pl.ds(r0, _RPW), pl.ds(off, _CHE)], mv, sem)

        def wait(dv, mv, sem):
            pltpu.make_async_copy(
                dst_hbm.at[pl.ds(e0, _CHE)], dv, sem).wait()
            pltpu.make_async_copy(
                msgt_hbm.at[pl.ds(r0, _RPW), pl.ds(e0, _CHE)], mv,
                sem).wait()

        def repair(idxs, vals):
            for r in range(_RPW):
                for _pass in range(16):
                    cur = plsc.load_gather(acc, [idxs[r]])
                    pending = vals[r] > cur
                    plsc.store_scatter(acc, [idxs[r]], vals[r],
                                       mask=pending)

        def process(dv, mv):
            @pl.loop(0, _CHE, step=16)
            def _(j):
                keys = dv[pl.ds(j, 16)]
                plsc.store_scatter(tmpn, [keys], io)
                idxs = [keys + (r * _N) for r in range(_RPW)]
                vals = [mv[r, pl.ds(j, 16)] for r in range(_RPW)]
                curs = [plsc.load_gather(acc, [idxs[r]])
                        for r in range(_RPW)]
                back = plsc.load_gather(tmpn, [keys])
                upds = [jnp.maximum(curs[r], vals[r]) for r in range(_RPW)]
                for r in range(_RPW):
                    plsc.store_scatter(acc, [idxs[r]], upds[r])
                has_dup = jnp.any(back != io)

                @pl.when(has_dup)
                def _():
                    repair(idxs, vals)

        start(0, dv0, mv0, sem0)

        @pl.loop(0, _NCHE // 2)
        def _(t):
            ci = t * 2
            start(ci + 1, dv1, mv1, sem1)
            wait(dv0, mv0, sem0)
            process(dv0, mv0)

            @pl.when(ci + 2 < _NCHE)
            def _():
                start(ci + 2, dv0, mv0, sem0)

            wait(dv1, mv1, sem1)
            process(dv1, mv1)

        base = (half * _D + r0) * _N
        pltpu.sync_copy(acc, ct_hbm.at[pl.ds(base, _RPW * _N)])

    return k(msgt, dst)


def _node_matmuls_kernel(h_ref, wa_ref, ba_ref, wb_ref, bb_ref, ah_ref, bh_ref):
    h = h_ref[...]
    ah_ref[...] = (
        jnp.dot(h, wa_ref[...], preferred_element_type=jnp.float32) + ba_ref[...]
    )
    bh_ref[...] = (
        jnp.dot(h, wb_ref[...], preferred_element_type=jnp.float32) + bb_ref[...]
    )


def _edge_fwd_kernel(e_ref, ga_ref, gsum_ref, wc_ref, bc_ref,
                     msg_ref, r_ref, stats_ref):
    i = pl.program_id(0)
    ce = (
        jnp.dot(e_ref[...], wc_ref[...], preferred_element_type=jnp.float32)
        + bc_ref[...]
    )
    e_ij = ce + gsum_ref[...]
    sig = jax.nn.sigmoid(e_ij)
    msg_ref[...] = jnp.maximum(sig * ga_ref[...], 0.0).T
    r = jnp.maximum(e_ij, 0.0)
    r_ref[...] = r

    @pl.when(i == 0)
    def _():
        stats_ref[...] = jnp.zeros_like(stats_ref)

    s = jnp.sum(r, axis=0, keepdims=True)
    s2 = jnp.sum(r * r, axis=0, keepdims=True)
    row = jax.lax.broadcasted_iota(jnp.int32, (8, _D), 0)
    upd = jnp.where(row == 0, s, 0.0) + jnp.where(row == 1, s2, 0.0)
    stats_ref[...] += upd


def _node_stage_kernel(h_ref, ct_ref, w1_ref, w2_ref, bap_ref, gh_ref, bh_ref,
                       hout_ref):
    h = h_ref[...]
    ct = jnp.maximum(ct_ref[0], ct_ref[1])
    c = ct.T
    c = jnp.where(jnp.isfinite(c), c, 0.0)
    bundle = (
        jnp.dot(h, w1_ref[...], preferred_element_type=jnp.float32)
        + jnp.dot(c, w2_ref[...], preferred_element_type=jnp.float32)
        + bap_ref[...]
    )
    norm = jnp.maximum(
        jnp.sqrt(jnp.sum(bundle * bundle, axis=1, keepdims=True)), 1e-12
    )
    hn = jnp.maximum(bundle / norm, 0.0)
    mu = jnp.mean(hn, axis=0, keepdims=True)
    var = jnp.mean(jnp.square(hn - mu), axis=0, keepdims=True)
    hn = gh_ref[...] * (hn - mu) / jnp.sqrt(var + 1e-5) + bh_ref[...]
    hout_ref[...] = h + hn


def _edge_out_kernel(e_ref, r_ref, mu_ref, isd_ref, be_ref, out_ref):
    out_ref[...] = (
        e_ref[...]
        + (r_ref[...] - mu_ref[...]) * isd_ref[...]
        + be_ref[...]
    )


def _row(v):
    return v.reshape(1, _D)


@jax.jit
def kernel(h, e, W_A, b_A, W_B, b_B, W_C, b_C, W_apply, b_apply,
           gamma_h, beta_h, gamma_e, beta_e, edge_index):
    src = edge_index[0]
    dst = edge_index[1]

    ah, bh = pl.pallas_call(
        _node_matmuls_kernel,
        out_shape=[
            jax.ShapeDtypeStruct((_N, _D), jnp.float32),
            jax.ShapeDtypeStruct((_N, _D), jnp.float32),
        ],
    )(h, W_A, _row(b_A), W_B, _row(b_B))

    ga, gsum = _sc_gather(ah, bh, src, dst)

    grid_e = _E // _BE
    msg, r, stats = pl.pallas_call(
        _edge_fwd_kernel,
        grid=(grid_e,),
        in_specs=[
            pl.BlockSpec((_BE, _D), lambda i: (i, 0)),
            pl.BlockSpec((_BE, _D), lambda i: (i, 0)),
            pl.BlockSpec((_BE, _D), lambda i: (i, 0)),
            pl.BlockSpec((_D, _D), lambda i: (0, 0)),
            pl.BlockSpec((1, _D), lambda i: (0, 0)),
        ],
        out_specs=[
            pl.BlockSpec((_D, _BE), lambda i: (0, i)),
            pl.BlockSpec((_BE, _D), lambda i: (i, 0)),
            pl.BlockSpec((8, _D), lambda i: (0, 0)),
        ],
        out_shape=[
            jax.ShapeDtypeStruct((_D, _E), jnp.float32),
            jax.ShapeDtypeStruct((_E, _D), jnp.float32),
            jax.ShapeDtypeStruct((8, _D), jnp.float32),
        ],
    )(e, ga, gsum, W_C, _row(b_C))

    ct = _sc_segmax(msg, dst).reshape(2, _D, _N)

    s = stats[0:1, :]
    s2 = stats[1:2, :]
    mu = s / _E
    var = s2 / _E - mu * mu
    isd = gamma_e.reshape(1, _D) / jnp.sqrt(var + 1e-5)

    e_out = pl.pallas_call(
        _edge_out_kernel,
        grid=(grid_e,),
        in_specs=[
            pl.BlockSpec((_BE, _D), lambda i: (i, 0)),
            pl.BlockSpec((_BE, _D), lambda i: (i, 0)),
            pl.BlockSpec((1, _D), lambda i: (0, 0)),
            pl.BlockSpec((1, _D), lambda i: (0, 0)),
            pl.BlockSpec((1, _D), lambda i: (0, 0)),
        ],
        out_specs=pl.BlockSpec((_BE, _D), lambda i: (i, 0)),
        out_shape=jax.ShapeDtypeStruct((_E, _D), jnp.float32),
    )(e, r, mu, isd, _row(beta_e))

    h_out = pl.pallas_call(
        _node_stage_kernel,
        out_shape=jax.ShapeDtypeStruct((_N, _D), jnp.float32),
    )(h, ct, W_apply[:_D], W_apply[_D:], _row(b_apply), _row(gamma_h),
      _row(beta_h))

    return (h_out, e_out)


# speedup vs baseline: 1.2800x; 1.2800x over previous
import dataclasses

import jax
import jax.numpy as jnp
from jax import lax
from jax.experimental import pallas as pl
from jax.experimental import pallas as pl_sc
from jax.experimental.pallas import tpu as pltpu
from jax.experimental.pallas import tpu_sc as plsc

_N = 10000
_E = 320000
_D = 128
_BE = 1280

_NW = 32
_PER_W = _E // _NW
_CH = 80
_NCH = _PER_W // _CH


_GW = 64


def _sc_gather(ah, bh, src, dst):
    mesh = plsc.VectorSubcoreMesh(core_axis_name="c", subcore_axis_name="s")
    out = jax.ShapeDtypeStruct((_E, _D), jnp.float32)

    src2 = src.reshape(_E // _GW, 1, _GW)
    dst2 = dst.reshape(_E // _GW, 1, _GW)

    @pl.kernel(
        mesh=mesh,
        out_type=[out, out, out],
        scratch_types=[
            pltpu.VMEM_SHARED((_N, _D), jnp.float32),
        ],
    )
    def k(ah_hbm, bh_hbm, src_hbm, dst_hbm, ga_hbm, gbs_hbm, gbd_hbm,
          bh_sh):
        sid = lax.axis_index("s")

        rows = 624
        pltpu.sync_copy(bh_hbm.at[pl.ds(sid * rows, rows)],
                        bh_sh.at[pl.ds(sid * rows, rows)])

        @pl.when(sid == 0)
        def _():
            pltpu.sync_copy(bh_hbm.at[pl.ds(16 * rows, _N - 16 * rows)],
                            bh_sh.at[pl.ds(16 * rows, _N - 16 * rows)])

        plsc.subcore_barrier()

        def body(src_v, dst_v, ga_v, gbs_v, gbd_v):
            pltpu.sync_copy(ah_hbm.at[src_v.at[0, 0]], ga_v)
            pltpu.sync_copy(bh_sh.at[src_v.at[0, 0]], gbs_v)
            pltpu.sync_copy(bh_sh.at[dst_v.at[0, 0]], gbd_v)

        pltpu.emit_pipeline(
            body,
            grid=(_E // _GW,),
            in_specs=[
                pl.BlockSpec((1, 1, _GW), lambda i: (i, 0, 0)),
                pl.BlockSpec((1, 1, _GW), lambda i: (i, 0, 0)),
            ],
            out_specs=[
                pl.BlockSpec((_GW, _D), lambda i: (i, 0)),
                pl.BlockSpec((_GW, _D), lambda i: (i, 0)),
                pl.BlockSpec((_GW, _D), lambda i: (i, 0)),
            ],
            core_axis_name=("c", "s"),
            dimension_semantics=(pltpu.PARALLEL,),
        )(src_hbm, dst_hbm, ga_hbm, gbs_hbm, gbd_hbm)

    return k(ah, bh, src2, dst2)


_CHE = 640
_EH = _E // 2
_NCHE = _EH // _CHE
_RPW = 8


def _sc_segmax(msgt, dst):
    mesh = plsc.VectorSubcoreMesh(core_axis_name="c", subcore_axis_name="s")

    cp = pltpu.CompilerParams()
    if "needs_layout_passes" in pltpu.CompilerParams.__dataclass_fields__:
        cp = dataclasses.replace(cp, needs_layout_passes=False)

    @pl.kernel(
        mesh=mesh,
        compiler_params=cp,
        out_type=jax.ShapeDtypeStruct((2 * _D * _N,), jnp.float32),
        scratch_types=[
            pltpu.VMEM((_RPW * _N,), jnp.float32),
            pltpu.VMEM((_N,), jnp.int32),
            pltpu.VMEM((_CHE,), jnp.int32),
            pltpu.VMEM((_CHE,), jnp.int32),
            pltpu.VMEM((_RPW, _CHE), jnp.float32),
            pltpu.VMEM((_RPW, _CHE), jnp.float32),
            pltpu.SemaphoreType.DMA,
            pltpu.SemaphoreType.DMA,
        ],
    )
    def k(msgt_hbm, dst_hbm, ct_hbm, acc, tmpn, dv0, dv1, mv0, mv1,
          sem0, sem1):
        w = lax.axis_index("s") * 2 + lax.axis_index("c")
        half = w % 2
        g = w // 2
        r0 = g * _RPW
        e0 = half * _EH

        neg_inf = jnp.full((16,), -jnp.inf, dtype=jnp.float32)
        io = lax.iota(jnp.int32, 16)
        io16 = io + 16

        @pl.loop(0, _RPW * _N, step=16)
        def _(cc):
            acc[pl.ds(cc, 16)] = neg_inf

        def start(ci, dv, mv, sem):
            off = e0 + ci * _CHE
            pltpu.async_copy(dst_hbm.at[pl.ds(off, _CHE)], dv, sem)
            pltpu.async_copy(
                msgt_hbm.at[pl.ds(r0, _RPW), pl.ds(off, _CHE)], mv, sem)

        def wait(dv, mv, sem):
            pltpu.make_async_copy(
                dst_hbm.at[pl.ds(e0, _CHE)], dv, sem).wait()
            pltpu.make_async_copy(
                msgt_hbm.at[pl.ds(r0, _RPW), pl.ds(e0, _CHE)], mv,
                sem).wait()

        def repair(idxs, vals):
            for r in range(_RPW):
                for _pass in range(16):
                    cur = plsc.load_gather(acc, [idxs[r]])
                    pending = vals[r] > cur
                    plsc.store_scatter(acc, [idxs[r]], vals[r],
                                       mask=pending)

        def process(dv, mv):
            @pl.loop(0, _CHE, step=16)
            def _(j):
                keys = dv[pl.ds(j, 16)]
                plsc.store_scatter(tmpn, [keys], io)
                idxs = [keys + (r * _N) for r in range(_RPW)]
                vals = [mv[r, pl.ds(j, 16)] for r in range(_RPW)]
                curs = [plsc.load_gather(acc, [idxs[r]])
                        for r in range(_RPW)]
                back = plsc.load_gather(tmpn, [keys])
                upds = [jnp.maximum(curs[r], vals[r]) for r in range(_RPW)]
                for r in range(_RPW):
                    plsc.store_scatter(acc, [idxs[r]], upds[r])
                has_dup = jnp.any(back != io)

                @pl.when(has_dup)
                def _():
                    repair(idxs, vals)

        start(0, dv0, mv0, sem0)

        @pl.loop(0, _NCHE // 2)
        def _(t):
            ci = t * 2
            start(ci + 1, dv1, mv1, sem1)
            wait(dv0, mv0, sem0)
            process(dv0, mv0)

            @pl.when(ci + 2 < _NCHE)
            def _():
                start(ci + 2, dv0, mv0, sem0)

            wait(dv1, mv1, sem1)
            process(dv1, mv1)

        base = (half * _D + r0) * _N
        pltpu.sync_copy(acc, ct_hbm.at[pl.ds(base, _RPW * _N)])

    return k(msgt, dst)


def _node_matmuls_kernel(h_ref, wa_ref, ba_ref, wb_ref, bb_ref, ah_ref, bh_ref):
    h = h_ref[...]
    ah_ref[...] = (
        jnp.dot(h, wa_ref[...], preferred_element_type=jnp.float32) + ba_ref[...]
    )
    bh_ref[...] = (
        jnp.dot(h, wb_ref[...], preferred_element_type=jnp.float32) + bb_ref[...]
    )


def _edge_fwd_kernel(e_ref, ga_ref, gbs_ref, gbd_ref, wc_ref, bc_ref,
                     msg_ref, r_ref, stats_ref):
    i = pl.program_id(0)
    ce = (
        jnp.dot(e_ref[...], wc_ref[...], preferred_element_type=jnp.float32)
        + bc_ref[...]
    )
    e_ij = ce + gbs_ref[...] + gbd_ref[...]
    sig = jax.nn.sigmoid(e_ij)
    msg_ref[...] = jnp.maximum(sig * ga_ref[...], 0.0).T
    r = jnp.maximum(e_ij, 0.0)
    r_ref[...] = r

    @pl.when(i == 0)
    def _():
        stats_ref[...] = jnp.zeros_like(stats_ref)

    s = jnp.sum(r, axis=0, keepdims=True)
    s2 = jnp.sum(r * r, axis=0, keepdims=True)
    row = jax.lax.broadcasted_iota(jnp.int32, (8, _D), 0)
    upd = jnp.where(row == 0, s, 0.0) + jnp.where(row == 1, s2, 0.0)
    stats_ref[...] += upd


def _node_stage_kernel(h_ref, ct_ref, w1_ref, w2_ref, bap_ref, gh_ref, bh_ref,
                       hout_ref):
    h = h_ref[...]
    ct = jnp.maximum(ct_ref[0], ct_ref[1])
    c = ct.T
    c = jnp.where(jnp.isfinite(c), c, 0.0)
    bundle = (
        jnp.dot(h, w1_ref[...], preferred_element_type=jnp.float32)
        + jnp.dot(c, w2_ref[...], preferred_element_type=jnp.float32)
        + bap_ref[...]
    )
    norm = jnp.maximum(
        jnp.sqrt(jnp.sum(bundle * bundle, axis=1, keepdims=True)), 1e-12
    )
    hn = jnp.maximum(bundle / norm, 0.0)
    mu = jnp.mean(hn, axis=0, keepdims=True)
    var = jnp.mean(jnp.square(hn - mu), axis=0, keepdims=True)
    hn = gh_ref[...] * (hn - mu) / jnp.sqrt(var + 1e-5) + bh_ref[...]
    hout_ref[...] = h + hn


def _edge_out_kernel(e_ref, r_ref, mu_ref, isd_ref, be_ref, out_ref):
    out_ref[...] = (
        e_ref[...]
        + (r_ref[...] - mu_ref[...]) * isd_ref[...]
        + be_ref[...]
    )


def _row(v):
    return v.reshape(1, _D)


@jax.jit
def kernel(h, e, W_A, b_A, W_B, b_B, W_C, b_C, W_apply, b_apply,
           gamma_h, beta_h, gamma_e, beta_e, edge_index):
    src = edge_index[0]
    dst = edge_index[1]

    ah, bh = pl.pallas_call(
        _node_matmuls_kernel,
        out_shape=[
            jax.ShapeDtypeStruct((_N, _D), jnp.float32),
            jax.ShapeDtypeStruct((_N, _D), jnp.float32),
        ],
    )(h, W_A, _row(b_A), W_B, _row(b_B))

    ga, gbs, gbd = _sc_gather(ah, bh, src, dst)

    grid_e = _E // _BE
    msg, r, stats = pl.pallas_call(
        _edge_fwd_kernel,
        grid=(grid_e,),
        in_specs=[
            pl.BlockSpec((_BE, _D), lambda i: (i, 0)),
            pl.BlockSpec((_BE, _D), lambda i: (i, 0)),
            pl.BlockSpec((_BE, _D), lambda i: (i, 0)),
            pl.BlockSpec((_BE, _D), lambda i: (i, 0)),
            pl.BlockSpec((_D, _D), lambda i: (0, 0)),
            pl.BlockSpec((1, _D), lambda i: (0, 0)),
        ],
        out_specs=[
            pl.BlockSpec((_D, _BE), lambda i: (0, i)),
            pl.BlockSpec((_BE, _D), lambda i: (i, 0)),
            pl.BlockSpec((8, _D), lambda i: (0, 0)),
        ],
        out_shape=[
            jax.ShapeDtypeStruct((_D, _E), jnp.float32),
            jax.ShapeDtypeStruct((_E, _D), jnp.float32),
            jax.ShapeDtypeStruct((8, _D), jnp.float32),
        ],
    )(e, ga, gbs, gbd, W_C, _row(b_C))

    ct = _sc_segmax(msg, dst).reshape(2, _D, _N)

    s = stats[0:1, :]
    s2 = stats[1:2, :]
    mu = s / _E
    var = s2 / _E - mu * mu
    isd = gamma_e.reshape(1, _D) / jnp.sqrt(var + 1e-5)

    e_out = pl.pallas_call(
        _edge_out_kernel,
        grid=(grid_e,),
        in_specs=[
            pl.BlockSpec((_BE, _D), lambda i: (i, 0)),
            pl.BlockSpec((_BE, _D), lambda i: (i, 0)),
            pl.BlockSpec((1, _D), lambda i: (0, 0)),
            pl.BlockSpec((1, _D), lambda i: (0, 0)),
            pl.BlockSpec((1, _D), lambda i: (0, 0)),
        ],
        out_specs=pl.BlockSpec((_BE, _D), lambda i: (i, 0)),
        out_shape=jax.ShapeDtypeStruct((_E, _D), jnp.float32),
    )(e, r, mu, isd, _row(beta_e))

    h_out = pl.pallas_call(
        _node_stage_kernel,
        out_shape=jax.ShapeDtypeStruct((_N, _D), jnp.float32),
    )(h, ct, W_apply[:_D], W_apply[_D:], _row(b_apply), _row(gamma_h),
      _row(beta_h))

    return (h_out, e_out)
